# baseline (speedup 1.0000x reference)
import functools

import jax
import jax.numpy as jnp
from jax import lax
from jax.experimental import pallas as pl
from jax.experimental.pallas import tpu as pltpu
from jax.experimental.pallas import tpu_sc as plsc

NW = 32
LANE = 128
SUB = 8
B_TILES_PER_W = 4
CHUNK_B = LANE
PITCH = LANE + 1


@functools.partial(jax.jit, static_argnums=(2, 3, 4))
def _embedding_gather(idx3, tab2, hist, bsz, d):
    jr_t = d // SUB
    b_tiles = bsz // LANE
    n_chunks = hist * B_TILES_PER_W
    mesh = plsc.VectorSubcoreMesh(core_axis_name="c", subcore_axis_name="s")

    @functools.partial(
        pl.kernel,
        mesh=mesh,
        out_type=jax.ShapeDtypeStruct((hist, jr_t, b_tiles, SUB, LANE), jnp.float32),
        compiler_params=pltpu.CompilerParams(
            use_tc_tiling_on_sc=False, needs_layout_passes=False
        ),
        scratch_types=[
            pltpu.VMEM((2, LANE), jnp.int32),
            pltpu.VMEM((2, CHUNK_B, 2 * d), jnp.float32),
            pltpu.VMEM((2, d, PITCH), jnp.float32),
            pltpu.SemaphoreType.DMA,
            pltpu.SemaphoreType.DMA,
            pltpu.SemaphoreType.DMA,
        ],
    )
    def body(
        idx_hbm, tab_hbm, out_hbm, idx_v, rows_v, rowsT_v,
        sem_i, sem_g, sem_o,
    ):
        wid = lax.axis_index("s") * 2 + lax.axis_index("c")
        ctile0 = wid * B_TILES_PER_W
        lane_iota = lax.iota(jnp.int32, 16)

        def h_of(t):
            return t // B_TILES_PER_W

        def ctile_of(t):
            return ctile0 + t % B_TILES_PER_W

        def idx_fetch(t, s):
            pltpu.async_copy(idx_hbm.at[h_of(t), ctile_of(t)], idx_v.at[s], sem_i)

        def idx_drain(s):
            pltpu.make_async_copy(idx_hbm.at[0, ctile0], idx_v.at[s], sem_i).wait()

        def gather_issue(s):
            pltpu.async_copy(tab_hbm.at[idx_v.at[s]], rows_v.at[s], sem_g)

        def gather_drain(s):
            pltpu.make_async_copy(
                tab_hbm.at[idx_v.at[s]], rows_v.at[s], sem_g
            ).wait()

        def wb_issue(t, s):
            for jt in range(jr_t):
                pltpu.async_copy(
                    rowsT_v.at[s, pl.ds(jt * SUB, SUB), pl.ds(0, LANE)],
                    out_hbm.at[h_of(t), jt, ctile_of(t)],
                    sem_o,
                )

        def wb_drain(t, s):
            for jt in range(jr_t):
                pltpu.make_async_copy(
                    rowsT_v.at[s, pl.ds(jt * SUB, SUB), pl.ds(0, LANE)],
                    out_hbm.at[h_of(t), jt, ctile_of(t)],
                    sem_o,
                ).wait()

        def transpose(s):
            rT = rowsT_v.at[s]
            jrows = [j0 * 16 + lane_iota for j0 in range(d // 16)]

            @plsc.parallel_loop(0, LANE, 1, unroll=2)
            def brow(b):
                bvec = jnp.full((16,), b, jnp.int32)
                for j0 in range(d // 16):
                    v = rows_v[s, b, pl.ds(j0 * 16, 16)]
                    plsc.store_scatter(rT, [jrows[j0], bvec], v)

        idx_fetch(0, 0)
        idx_fetch(1, 1)
        idx_drain(0)
        gather_issue(0)

        def step(t, carry):
            s = t % 2
            sn = (t + 1) % 2
            gather_drain(s)
            @pl.when(t + 2 < n_chunks)
            def _():
                idx_fetch(t + 2, s)

            @pl.when(t + 1 < n_chunks)
            def _():
                idx_drain(sn)
                gather_issue(sn)

            @pl.when(t >= 2)
            def _():
                wb_drain(t, s)

            transpose(s)
            wb_issue(t, s)
            return carry

        lax.fori_loop(0, n_chunks, step, 0)
        wb_drain(n_chunks - 2, 0)
        wb_drain(n_chunks - 1, 1)

    return body(idx3, tab2)


def kernel(x, embed_matrix):
    bsz, hist = x.shape
    v, d = embed_matrix.shape
    idx3 = jnp.transpose(x).reshape(hist, bsz // LANE, LANE).astype(jnp.int32)
    tab2 = jnp.pad(embed_matrix, ((0, 0), (0, d)))
    out5 = _embedding_gather(idx3, tab2, hist, bsz, d)
    return out5.transpose(2, 4, 0, 1, 3).reshape(bsz, hist, d)


# speedup vs baseline: 1.2078x; 1.2078x over previous
import functools

import jax
import jax.numpy as jnp
from jax import lax
from jax.experimental import pallas as pl
from jax.experimental.pallas import tpu as pltpu
from jax.experimental.pallas import tpu_sc as plsc

NW = 32
LANE = 128
SUB = 8
B_TILES_PER_W = 4
CHUNK_T = 2
CHUNK_B = CHUNK_T * LANE
PITCH = LANE + 1


@functools.partial(jax.jit, static_argnums=(2, 3, 4))
def _embedding_gather(idx3, table, hist, bsz, d):
    jr_t = d // SUB
    b_tiles = bsz // LANE
    n_chunks = hist * (B_TILES_PER_W // CHUNK_T)
    mesh = plsc.VectorSubcoreMesh(core_axis_name="c", subcore_axis_name="s")

    @functools.partial(
        pl.kernel,
        mesh=mesh,
        out_type=jax.ShapeDtypeStruct((hist, jr_t, b_tiles, SUB, LANE), jnp.float32),
        compiler_params=pltpu.CompilerParams(
            use_tc_tiling_on_sc=False, needs_layout_passes=False
        ),
        scratch_types=[
            pltpu.VMEM((2, CHUNK_T, LANE), jnp.int32),
            pltpu.VMEM((2, CHUNK_T, LANE), jnp.int32),
            pltpu.VMEM((2, CHUNK_B, d), jnp.float32),
            pltpu.VMEM((2, CHUNK_T, d, PITCH), jnp.float32),
            pltpu.SemaphoreType.DMA,
            pltpu.SemaphoreType.DMA,
            pltpu.SemaphoreType.DMA,
        ],
    )
    def body(
        idx_hbm, tab_hbm, out_hbm, idx_v, idx2_v, rows_v, rowsT_v,
        sem_i, sem_g, sem_o,
    ):
        wid = lax.axis_index("s") * 2 + lax.axis_index("c")
        ctile0 = wid * B_TILES_PER_W
        lane_iota = lax.iota(jnp.int32, 16)

        def h_of(t):
            return t // 2

        def cbase_of(t):
            return ctile0 + (t % 2) * CHUNK_T

        def idx_fetch(t, s):
            pltpu.async_copy(
                idx_hbm.at[h_of(t), pl.ds(cbase_of(t), CHUNK_T)], idx_v.at[s], sem_i
            )

        def idx_drain(s):
            pltpu.make_async_copy(
                idx_hbm.at[0, pl.ds(ctile0, CHUNK_T)], idx_v.at[s], sem_i
            ).wait()

        def idx_double(s):
            for cc in range(CHUNK_T):
                for g in range(LANE // 16):
                    idx2_v[s, cc, pl.ds(g * 16, 16)] = (
                        idx_v[s, cc, pl.ds(g * 16, 16)] * 2
                    )

        def gathers_issue(s):
            for cc in range(CHUNK_T):
                pltpu.async_copy(
                    tab_hbm.at[idx2_v.at[s, cc]],
                    rows_v.at[s, pl.ds(cc * LANE, LANE)],
                    sem_g,
                )

        def gathers_drain(s):
            for cc in range(CHUNK_T):
                pltpu.make_async_copy(
                    tab_hbm.at[idx2_v.at[s, cc]],
                    rows_v.at[s, pl.ds(cc * LANE, LANE)],
                    sem_g,
                ).wait()

        def wb_issue(t, s):
            for cp in range(CHUNK_T):
                for jt in range(jr_t):
                    pltpu.async_copy(
                        rowsT_v.at[s, cp, pl.ds(jt * SUB, SUB), pl.ds(0, LANE)],
                        out_hbm.at[h_of(t), jt, cbase_of(t) + cp],
                        sem_o,
                    )

        def wb_drain(t, s):
            for cp in range(CHUNK_T):
                for jt in range(jr_t):
                    pltpu.make_async_copy(
                        rowsT_v.at[s, cp, pl.ds(jt * SUB, SUB), pl.ds(0, LANE)],
                        out_hbm.at[h_of(t), jt, cbase_of(t) + cp],
                        sem_o,
                    ).wait()

        def transpose(s):
            for cp in range(CHUNK_T):
                rT = rowsT_v.at[s, cp]
                jrows = [j0 * 16 + lane_iota for j0 in range(d // 16)]

                @plsc.parallel_loop(0, LANE, 1, unroll=2)
                def brow(b):
                    bvec = jnp.full((16,), b, jnp.int32)
                    for j0 in range(d // 16):
                        v = rows_v[s, cp * LANE + b, pl.ds(j0 * 16, 16)]
                        plsc.store_scatter(rT, [jrows[j0], bvec], v)

        idx_fetch(0, 0)
        idx_fetch(1, 1)
        idx_drain(0)
        idx_double(0)
        gathers_issue(0)

        def step(t, carry):
            s = t % 2
            sn = (t + 1) % 2
            gathers_drain(s)
            @pl.when(t + 2 < n_chunks)
            def _():
                idx_fetch(t + 2, s)

            @pl.when(t + 1 < n_chunks)
            def _():
                idx_drain(sn)
                idx_double(sn)
                gathers_issue(sn)

            @pl.when(t >= 2)
            def _():
                wb_drain(t, s)

            transpose(s)
            wb_issue(t, s)
            return carry

        lax.fori_loop(0, n_chunks, step, 0)
        wb_drain(n_chunks - 2, 0)
        wb_drain(n_chunks - 1, 1)

    return body(idx3, table)


def kernel(x, embed_matrix):
    bsz, hist = x.shape
    v, d = embed_matrix.shape
    idx3 = jnp.transpose(x).reshape(hist, bsz // LANE, LANE).astype(jnp.int32)
    tab2 = jnp.pad(embed_matrix, ((0, 0), (0, d))).reshape(2 * v, d)
    out5 = _embedding_gather(idx3, tab2, hist, bsz, d)
    return out5.transpose(2, 4, 0, 1, 3).reshape(bsz, hist, d)
